# Initial kernel scaffold; baseline (speedup 1.0000x reference)
import jax
import jax.numpy as jnp
from jax.experimental import pallas as pl


def kernel(x, edge_index, edge_attr, W1, b1, W2, b2, Ws, bs, U1, c1, gamma, beta, U2, c2):
    raise NotImplementedError("write your pallas kernel here")


# speedup vs baseline: 2.8090x; 2.8090x over previous
import functools

import jax
import jax.numpy as jnp
from jax import lax
from jax.experimental import pallas as pl
from jax.experimental.pallas import tpu as pltpu
from jax.experimental.pallas import tpu_sc as plsc

_NC = 2
_NS = 16
_NW = _NC * _NS
_L = 16


def _proj_body(x_ref, w1_ref, b1_ref, ps_ref, pd_ref):
    h = ps_ref.shape[1]
    x = x_ref[...]
    ps_ref[...] = jnp.dot(x, w1_ref[:h, :], preferred_element_type=jnp.float32)
    pd_ref[...] = (
        jnp.dot(x, w1_ref[h : 2 * h, :], preferred_element_type=jnp.float32)
        + b1_ref[...]
    )


def _proj(x, w1, b1):
    n, h = x.shape
    return pl.pallas_call(
        _proj_body,
        out_shape=[
            jax.ShapeDtypeStruct((n, h), jnp.float32),
            jax.ShapeDtypeStruct((n, h), jnp.float32),
        ],
    )(x, w1, b1.reshape(1, h))


def _make_gather(n, e, h, chunk):
    epw = e // _NW
    n_chunks = epw // chunk
    mesh = plsc.VectorSubcoreMesh(core_axis_name="c", subcore_axis_name="s")

    @functools.partial(
        pl.kernel,
        out_type=jax.ShapeDtypeStruct((e, h), jnp.float32),
        mesh=mesh,
        scratch_types=[
            pltpu.VMEM((chunk,), jnp.int32),
            pltpu.VMEM((chunk,), jnp.int32),
            pltpu.VMEM((chunk, h), jnp.float32),
            pltpu.VMEM((chunk, h), jnp.float32),
            pltpu.SemaphoreType.DMA,
            pltpu.SemaphoreType.DMA,
        ],
    )
    def gather_k(ps_hbm, pd_hbm, src_hbm, dst_hbm, out_hbm,
                 idxs_v, idxd_v, rows_s, rows_d, sem_s, sem_d):
        wid = lax.axis_index("s") * _NC + lax.axis_index("c")
        base = wid * epw

        def body(g, carry):
            off = base + g * chunk
            pltpu.sync_copy(src_hbm.at[pl.ds(off, chunk)], idxs_v)
            pltpu.sync_copy(dst_hbm.at[pl.ds(off, chunk)], idxd_v)
            cs = pltpu.async_copy(ps_hbm.at[idxs_v], rows_s, sem_s)
            cd = pltpu.async_copy(pd_hbm.at[idxd_v], rows_d, sem_d)
            cs.wait()
            cd.wait()

            def radd(r, c2):
                for j in range(h // _L):
                    sl = pl.ds(j * _L, _L)
                    rows_s[r, sl] = rows_s[r, sl] + rows_d[r, sl]
                return c2

            lax.fori_loop(0, chunk, radd, 0)
            pltpu.sync_copy(rows_s, out_hbm.at[pl.ds(off, chunk)])
            return carry

        lax.fori_loop(0, n_chunks, body, 0)

    return gather_k


def _edge_mlp_body(g_ref, ea_ref, w1c_ref, w2_ref, b2_ref, ws_ref, bs_ref,
                   dnew_ref, gated_ref):
    g = g_ref[...]
    ea = ea_ref[...]
    h1 = jnp.maximum(
        g + jnp.dot(ea, w1c_ref[...], preferred_element_type=jnp.float32), 0.0
    )
    z = jnp.dot(h1, w2_ref[...], preferred_element_type=jnp.float32) + b2_ref[...]
    msg = z * jax.nn.sigmoid(z)
    dnew_ref[...] = ea + msg
    ew = jax.nn.sigmoid(
        jnp.dot(msg, ws_ref[...], preferred_element_type=jnp.float32) + bs_ref[...]
    )
    gated_ref[...] = msg * ew


def _edge_mlp(g, ea, w1c, w2, b2, ws, bs, block_e):
    e, h = ea.shape
    grid = (e // block_e,)
    blk = lambda i: (i, 0)
    fix = lambda i: (0, 0)
    return pl.pallas_call(
        _edge_mlp_body,
        grid=grid,
        in_specs=[
            pl.BlockSpec((block_e, h), blk),
            pl.BlockSpec((block_e, h), blk),
            pl.BlockSpec((h, h), fix),
            pl.BlockSpec((h, h), fix),
            pl.BlockSpec((1, h), fix),
            pl.BlockSpec((h, 1), fix),
            pl.BlockSpec((1, 1), fix),
        ],
        out_specs=[
            pl.BlockSpec((block_e, h), blk),
            pl.BlockSpec((block_e, h), blk),
        ],
        out_shape=[
            jax.ShapeDtypeStruct((e, h), jnp.float32),
            jax.ShapeDtypeStruct((e, h), jnp.float32),
        ],
    )(g, ea, w1c, w2, b2.reshape(1, h), ws, bs.reshape(1, 1))


def _make_scatter(n, e, h, chunk):
    epw = e // _NW
    n_chunks = epw // chunk
    zch = 80
    n_zchunks = n // zch
    mesh = plsc.VectorSubcoreMesh(core_axis_name="c", subcore_axis_name="s")

    @functools.partial(
        pl.kernel,
        out_type=jax.ShapeDtypeStruct((_NC, n, h), jnp.float32),
        mesh=mesh,
        scratch_types=[
            pltpu.VMEM((chunk,), jnp.int32),
            pltpu.VMEM((chunk, h), jnp.float32),
            pltpu.VMEM((zch, h), jnp.float32),
            pltpu.VMEM_SHARED((n, h), jnp.float32),
        ],
    )
    def scatter_k(gated_hbm, dst_hbm, out_hbm, idx_v, rows_v, zbuf, acc_sh):
        c = lax.axis_index("c")
        s = lax.axis_index("s")
        wid = s * _NC + c
        base = wid * epw
        zero = jnp.zeros((_L,), jnp.float32)

        def zr(r, carry):
            for j in range(h // _L):
                zbuf[r, pl.ds(j * _L, _L)] = zero
            return carry

        lax.fori_loop(0, zch, zr, 0)

        n_mine = (n_zchunks - s + _NS - 1) // _NS

        def zdma(i, carry):
            q = s + i * _NS
            pltpu.sync_copy(zbuf, acc_sh.at[pl.ds(q * zch, zch)])
            return carry

        lax.fori_loop(0, n_mine, zdma, 0)
        plsc.subcore_barrier()

        def body(g, carry):
            off = base + g * chunk
            pltpu.sync_copy(dst_hbm.at[pl.ds(off, chunk)], idx_v)
            pltpu.sync_copy(gated_hbm.at[pl.ds(off, chunk)], rows_v)
            pltpu.sync_copy(rows_v, acc_sh.at[idx_v], add=True)
            return carry

        lax.fori_loop(0, n_chunks, body, 0)
        plsc.subcore_barrier()

        def odma(i, carry):
            r0 = (s + i * _NS) * zch
            pltpu.sync_copy(acc_sh.at[pl.ds(r0, zch)],
                            out_hbm.at[c, pl.ds(r0, zch)])
            return carry

        lax.fori_loop(0, n_mine, odma, 0)

    return scatter_k


def _update_body(m_ref, x_ref, u1_ref, c1_ref, gam_ref, bet_ref, u2_ref,
                 c2_ref, out_ref):
    x = x_ref[...]
    u_in = m_ref[0] + m_ref[1] + x
    u1 = jnp.maximum(
        jnp.dot(u_in, u1_ref[...], preferred_element_type=jnp.float32)
        + c1_ref[...],
        0.0,
    )
    mean = jnp.mean(u1, axis=0, keepdims=True)
    var = jnp.mean((u1 - mean) ** 2, axis=0, keepdims=True)
    u1n = (u1 - mean) * lax.rsqrt(var + 1e-5) * gam_ref[...] + bet_ref[...]
    out_ref[...] = (
        jnp.dot(u1n, u2_ref[...], preferred_element_type=jnp.float32)
        + c2_ref[...]
        + x
    )


def _update(m, x, u1, c1, gamma, beta, u2, c2):
    n, h = x.shape
    return pl.pallas_call(
        _update_body,
        out_shape=jax.ShapeDtypeStruct((n, h), jnp.float32),
    )(m, x, u1, c1.reshape(1, h), gamma.reshape(1, h), beta.reshape(1, h),
      u2, c2.reshape(1, h))


def kernel(x, edge_index, edge_attr, W1, b1, W2, b2, Ws, bs,
           U1, c1, gamma, beta, U2, c2):
    n, h = x.shape
    e = edge_attr.shape[0]
    src = edge_index[0]
    dst = edge_index[1]

    ps, pd = _proj(x, W1, b1)
    g = _make_gather(n, e, h, chunk=80)(ps, pd, src, dst)
    d_new, gated = _edge_mlp(g, edge_attr, W1[2 * h :, :], W2, b2, Ws, bs,
                             block_e=2000)
    m_part = _make_scatter(n, e, h, chunk=80)(gated, dst)
    feat_out = _update(m_part, x, U1, c1, gamma, beta, U2, c2)
    return feat_out, d_new


# speedup vs baseline: 4.5517x; 1.6204x over previous
import functools

import jax
import jax.numpy as jnp
from jax import lax
from jax.experimental import pallas as pl
from jax.experimental.pallas import tpu as pltpu
from jax.experimental.pallas import tpu_sc as plsc

_NC = 2
_NS = 16
_NW = _NC * _NS
_L = 16


def _proj_body(x_ref, w1_ref, b1_ref, ps_ref, pd_ref):
    h = ps_ref.shape[1]
    x = x_ref[...]
    ps_ref[...] = jnp.dot(x, w1_ref[:h, :], preferred_element_type=jnp.float32)
    pd_ref[...] = (
        jnp.dot(x, w1_ref[h : 2 * h, :], preferred_element_type=jnp.float32)
        + b1_ref[...]
    )


def _proj(x, w1, b1):
    n, h = x.shape
    return pl.pallas_call(
        _proj_body,
        out_shape=[
            jax.ShapeDtypeStruct((n, h), jnp.float32),
            jax.ShapeDtypeStruct((n, h), jnp.float32),
        ],
    )(x, w1, b1.reshape(1, h))


def _make_gather(n, e, h, chunk):
    epw = e // _NW
    n_chunks = epw // chunk
    mesh = plsc.VectorSubcoreMesh(core_axis_name="c", subcore_axis_name="s")

    @functools.partial(
        pl.kernel,
        out_type=jax.ShapeDtypeStruct((e, h), jnp.float32),
        mesh=mesh,
        scratch_types=[
            pltpu.VMEM((n_chunks, chunk), jnp.int32),
            pltpu.VMEM((n_chunks, chunk), jnp.int32),
            pltpu.VMEM((chunk, h), jnp.float32),
            pltpu.VMEM((chunk, h), jnp.float32),
            pltpu.VMEM((chunk, h), jnp.float32),
            pltpu.VMEM((chunk, h), jnp.float32),
            pltpu.VMEM((chunk, h), jnp.float32),
            pltpu.VMEM((chunk, h), jnp.float32),
            pltpu.SemaphoreType.DMA,
            pltpu.SemaphoreType.DMA,
            pltpu.SemaphoreType.DMA,
            pltpu.SemaphoreType.DMA,
        ],
    )
    def gather_k(ps_hbm, pd_hbm, src_hbm, dst_hbm, out_hbm,
                 idxs_v, idxd_v, rs0, rs1, rd0, rd1, ro0, ro1,
                 sg0, sg1, so0, so1):
        wid = lax.axis_index("s") * _NC + lax.axis_index("c")
        base = wid * epw
        pltpu.sync_copy(src_hbm.at[wid], idxs_v)
        pltpu.sync_copy(dst_hbm.at[wid], idxd_v)

        RS, RD, RO = (rs0, rs1), (rd0, rd1), (ro0, ro1)
        SG, SO = (sg0, sg1), (so0, so1)

        def start_gather(g, b):
            pltpu.async_copy(ps_hbm.at[idxs_v.at[g]], RS[b], SG[b])
            pltpu.async_copy(pd_hbm.at[idxd_v.at[g]], RD[b], SG[b])

        def wait_gather(b):
            pltpu.make_async_copy(ps_hbm.at[idxs_v.at[0]], RS[b], SG[b]).wait()
            pltpu.make_async_copy(pd_hbm.at[idxd_v.at[0]], RD[b], SG[b]).wait()

        def add_rows(b):
            def radd(r, c2):
                for j in range(h // _L):
                    sl = pl.ds(j * _L, _L)
                    RO[b][r, sl] = RS[b][r, sl] + RD[b][r, sl]
                return c2

            lax.fori_loop(0, chunk, radd, 0)

        def start_out(g, b):
            pltpu.async_copy(RO[b], out_hbm.at[pl.ds(base + g * chunk, chunk)],
                             SO[b])

        def wait_out(b):
            pltpu.make_async_copy(
                RO[b], out_hbm.at[pl.ds(base, chunk)], SO[b]).wait()

        for b in range(2):
            start_gather(b, b)

        def pair(p, carry):
            for b in range(2):
                g = p * 2 + b
                wait_gather(b)

                @pl.when(g >= 2)
                def _():
                    wait_out(b)

                add_rows(b)
                start_out(g, b)

                @pl.when(g + 2 < n_chunks)
                def _():
                    start_gather(g + 2, b)

            return carry

        n_pairs = n_chunks // 2
        lax.fori_loop(0, n_pairs, pair, 0)
        if n_chunks % 2:
            g = n_chunks - 1
            wait_gather(0)
            wait_out(0)
            add_rows(0)
            start_out(g, 0)
            wait_out(1)
            wait_out(0)
        else:
            wait_out(0)
            wait_out(1)

    return gather_k


def _edge_mlp_body(g_ref, ea_ref, w1c_ref, w2_ref, b2_ref, ws_ref, bs_ref,
                   dnew_ref, gated_ref):
    g = g_ref[...]
    ea = ea_ref[...]
    h1 = jnp.maximum(
        g + jnp.dot(ea, w1c_ref[...], preferred_element_type=jnp.float32), 0.0
    )
    z = jnp.dot(h1, w2_ref[...], preferred_element_type=jnp.float32) + b2_ref[...]
    msg = z * jax.nn.sigmoid(z)
    dnew_ref[...] = ea + msg
    ew = jax.nn.sigmoid(
        jnp.dot(msg, ws_ref[...], preferred_element_type=jnp.float32) + bs_ref[...]
    )
    gated_ref[...] = msg * ew


def _edge_mlp(g, ea, w1c, w2, b2, ws, bs, block_e):
    e, h = ea.shape
    grid = (e // block_e,)
    blk = lambda i: (i, 0)
    fix = lambda i: (0, 0)
    return pl.pallas_call(
        _edge_mlp_body,
        grid=grid,
        in_specs=[
            pl.BlockSpec((block_e, h), blk),
            pl.BlockSpec((block_e, h), blk),
            pl.BlockSpec((h, h), fix),
            pl.BlockSpec((h, h), fix),
            pl.BlockSpec((1, h), fix),
            pl.BlockSpec((h, 1), fix),
            pl.BlockSpec((1, 1), fix),
        ],
        out_specs=[
            pl.BlockSpec((block_e, h), blk),
            pl.BlockSpec((block_e, h), blk),
        ],
        out_shape=[
            jax.ShapeDtypeStruct((e, h), jnp.float32),
            jax.ShapeDtypeStruct((e, h), jnp.float32),
        ],
    )(g, ea, w1c, w2, b2.reshape(1, h), ws, bs.reshape(1, 1))


def _make_scatter(n, e, h, chunk):
    epw = e // _NW
    n_chunks = epw // chunk
    zch = 80
    n_zchunks = n // zch
    mesh = plsc.VectorSubcoreMesh(core_axis_name="c", subcore_axis_name="s")

    @functools.partial(
        pl.kernel,
        out_type=jax.ShapeDtypeStruct((_NC, n, h), jnp.float32),
        mesh=mesh,
        scratch_types=[
            pltpu.VMEM((n_chunks, chunk), jnp.int32),
            pltpu.VMEM((chunk, h), jnp.float32),
            pltpu.VMEM((chunk, h), jnp.float32),
            pltpu.VMEM((zch, h), jnp.float32),
            pltpu.VMEM_SHARED((n, h), jnp.float32),
            pltpu.SemaphoreType.DMA,
            pltpu.SemaphoreType.DMA,
        ],
    )
    def scatter_k(gated_hbm, dst_hbm, out_hbm, idx_v, rv0, rv1, zbuf, acc_sh,
                  sl0, sl1):
        c = lax.axis_index("c")
        s = lax.axis_index("s")
        wid = s * _NC + c
        base = wid * epw
        zero = jnp.zeros((_L,), jnp.float32)

        def zr(r, carry):
            for j in range(h // _L):
                zbuf[r, pl.ds(j * _L, _L)] = zero
            return carry

        lax.fori_loop(0, zch, zr, 0)

        n_mine = (n_zchunks - s + _NS - 1) // _NS

        def zdma(i, carry):
            q = s + i * _NS
            pltpu.sync_copy(zbuf, acc_sh.at[pl.ds(q * zch, zch)])
            return carry

        lax.fori_loop(0, n_mine, zdma, 0)
        pltpu.sync_copy(dst_hbm.at[wid], idx_v)
        plsc.subcore_barrier()

        RV = (rv0, rv1)
        SL = (sl0, sl1)

        def start_load(g, b):
            pltpu.async_copy(
                gated_hbm.at[pl.ds(base + g * chunk, chunk)], RV[b], SL[b])

        def wait_load(b):
            pltpu.make_async_copy(
                gated_hbm.at[pl.ds(base, chunk)], RV[b], SL[b]).wait()

        for b in range(2):
            start_load(b, b)

        def pair(p, carry):
            for b in range(2):
                g = p * 2 + b
                wait_load(b)
                pltpu.sync_copy(RV[b], acc_sh.at[idx_v.at[g]], add=True)

                @pl.when(g + 2 < n_chunks)
                def _():
                    start_load(g + 2, b)

            return carry

        lax.fori_loop(0, n_chunks // 2, pair, 0)
        if n_chunks % 2:
            g = n_chunks - 1
            wait_load(0)
            pltpu.sync_copy(RV[0], acc_sh.at[idx_v.at[g]], add=True)
        plsc.subcore_barrier()

        def odma(i, carry):
            r0 = (s + i * _NS) * zch
            pltpu.sync_copy(acc_sh.at[pl.ds(r0, zch)],
                            out_hbm.at[c, pl.ds(r0, zch)])
            return carry

        lax.fori_loop(0, n_mine, odma, 0)

    return scatter_k


def _update_body(m_ref, x_ref, u1_ref, c1_ref, gam_ref, bet_ref, u2_ref,
                 c2_ref, out_ref):
    x = x_ref[...]
    u_in = m_ref[0] + m_ref[1] + x
    u1 = jnp.maximum(
        jnp.dot(u_in, u1_ref[...], preferred_element_type=jnp.float32)
        + c1_ref[...],
        0.0,
    )
    mean = jnp.mean(u1, axis=0, keepdims=True)
    var = jnp.mean((u1 - mean) ** 2, axis=0, keepdims=True)
    u1n = (u1 - mean) * lax.rsqrt(var + 1e-5) * gam_ref[...] + bet_ref[...]
    out_ref[...] = (
        jnp.dot(u1n, u2_ref[...], preferred_element_type=jnp.float32)
        + c2_ref[...]
        + x
    )


def _update(m, x, u1, c1, gamma, beta, u2, c2):
    n, h = x.shape
    return pl.pallas_call(
        _update_body,
        out_shape=jax.ShapeDtypeStruct((n, h), jnp.float32),
    )(m, x, u1, c1.reshape(1, h), gamma.reshape(1, h), beta.reshape(1, h),
      u2, c2.reshape(1, h))


def kernel(x, edge_index, edge_attr, W1, b1, W2, b2, Ws, bs,
           U1, c1, gamma, beta, U2, c2):
    n, h = x.shape
    e = edge_attr.shape[0]
    chunk = 80
    src3 = edge_index[0].reshape(_NW, -1, chunk)
    dst3 = edge_index[1].reshape(_NW, -1, chunk)

    ps, pd = _proj(x, W1, b1)
    g = _make_gather(n, e, h, chunk)(ps, pd, src3, dst3)
    d_new, gated = _edge_mlp(g, edge_attr, W1[2 * h :, :], W2, b2, Ws, bs,
                             block_e=2000)
    m_part = _make_scatter(n, e, h, chunk)(gated, dst3)
    feat_out = _update(m_part, x, U1, c1, gamma, beta, U2, c2)
    return feat_out, d_new


# speedup vs baseline: 4.8158x; 1.0580x over previous
import functools

import jax
import jax.numpy as jnp
from jax import lax
from jax.experimental import pallas as pl
from jax.experimental.pallas import tpu as pltpu
from jax.experimental.pallas import tpu_sc as plsc

_NC = 2
_NS = 16
_NW = _NC * _NS
_L = 16


def _proj_body(x_ref, w1_ref, b1_ref, ps_ref, pd_ref):
    h = ps_ref.shape[1]
    x = x_ref[...]
    ps_ref[...] = jnp.dot(x, w1_ref[:h, :], preferred_element_type=jnp.float32)
    pd_ref[...] = (
        jnp.dot(x, w1_ref[h : 2 * h, :], preferred_element_type=jnp.float32)
        + b1_ref[...]
    )


def _proj(x, w1, b1):
    n, h = x.shape
    return pl.pallas_call(
        _proj_body,
        out_shape=[
            jax.ShapeDtypeStruct((n, h), jnp.float32),
            jax.ShapeDtypeStruct((n, h), jnp.float32),
        ],
    )(x, w1, b1.reshape(1, h))


def _make_gather(n, e, h, chunk):
    epw = e // _NW
    n_chunks = epw // chunk
    mesh = plsc.VectorSubcoreMesh(core_axis_name="c", subcore_axis_name="s")

    @functools.partial(
        pl.kernel,
        out_type=jax.ShapeDtypeStruct((e, h), jnp.float32),
        mesh=mesh,
        scratch_types=[
            pltpu.VMEM((n_chunks, chunk), jnp.int32),
            pltpu.VMEM((n_chunks, chunk), jnp.int32),
            pltpu.VMEM((chunk, h), jnp.float32),
            pltpu.VMEM((chunk, h), jnp.float32),
            pltpu.VMEM((chunk, h), jnp.float32),
            pltpu.VMEM((chunk, h), jnp.float32),
            pltpu.VMEM((chunk, h), jnp.float32),
            pltpu.VMEM((chunk, h), jnp.float32),
            pltpu.SemaphoreType.DMA,
            pltpu.SemaphoreType.DMA,
            pltpu.SemaphoreType.DMA,
            pltpu.SemaphoreType.DMA,
        ],
    )
    def gather_k(ps_hbm, pd_hbm, src_hbm, dst_hbm, out_hbm,
                 idxs_v, idxd_v, rs0, rs1, rd0, rd1, ro0, ro1,
                 sg0, sg1, so0, so1):
        wid = lax.axis_index("s") * _NC + lax.axis_index("c")
        base = wid * epw
        pltpu.sync_copy(src_hbm.at[wid], idxs_v)
        pltpu.sync_copy(dst_hbm.at[wid], idxd_v)

        RS, RD, RO = (rs0, rs1), (rd0, rd1), (ro0, ro1)
        SG, SO = (sg0, sg1), (so0, so1)

        def start_gather(g, b):
            pltpu.async_copy(ps_hbm.at[idxs_v.at[g]], RS[b], SG[b])
            pltpu.async_copy(pd_hbm.at[idxd_v.at[g]], RD[b], SG[b])

        def wait_gather(b):
            pltpu.make_async_copy(ps_hbm.at[idxs_v.at[0]], RS[b], SG[b]).wait()
            pltpu.make_async_copy(pd_hbm.at[idxd_v.at[0]], RD[b], SG[b]).wait()

        def add_rows(b):
            def radd(r, c2):
                for j in range(h // _L):
                    sl = pl.ds(j * _L, _L)
                    RO[b][r, sl] = RS[b][r, sl] + RD[b][r, sl]
                return c2

            lax.fori_loop(0, chunk, radd, 0)

        def start_out(g, b):
            pltpu.async_copy(RO[b], out_hbm.at[pl.ds(base + g * chunk, chunk)],
                             SO[b])

        def wait_out(b):
            pltpu.make_async_copy(
                RO[b], out_hbm.at[pl.ds(base, chunk)], SO[b]).wait()

        for b in range(2):
            start_gather(b, b)

        def pair(p, carry):
            for b in range(2):
                g = p * 2 + b
                wait_gather(b)

                @pl.when(g >= 2)
                def _():
                    wait_out(b)

                add_rows(b)
                start_out(g, b)

                @pl.when(g + 2 < n_chunks)
                def _():
                    start_gather(g + 2, b)

            return carry

        n_pairs = n_chunks // 2
        lax.fori_loop(0, n_pairs, pair, 0)
        if n_chunks % 2:
            g = n_chunks - 1
            wait_gather(0)
            wait_out(0)
            add_rows(0)
            start_out(g, 0)
            wait_out(1)
            wait_out(0)
        else:
            wait_out(0)
            wait_out(1)

    return gather_k


def _edge_mlp_body(g_ref, ea_ref, w1c_ref, w2_ref, b2_ref, ws_ref, bs_ref,
                   dnew_ref, gated_ref):
    g = g_ref[...]
    ea = ea_ref[...]
    h1 = jnp.maximum(
        g + jnp.dot(ea, w1c_ref[...], preferred_element_type=jnp.float32), 0.0
    )
    z = jnp.dot(h1, w2_ref[...], preferred_element_type=jnp.float32) + b2_ref[...]
    msg = z * jax.nn.sigmoid(z)
    dnew_ref[...] = ea + msg
    ew = jax.nn.sigmoid(
        jnp.dot(msg, ws_ref[...], preferred_element_type=jnp.float32) + bs_ref[...]
    )
    gated_ref[...] = msg * ew


def _edge_mlp_half(g, ea, w1c, w2, b2, ws, bs, block_e, half, dnew_prev):
    eh, h = g.shape
    e = ea.shape[0]
    nb = eh // block_e
    off = half * nb
    blk = lambda i: (i, 0)
    ebk = lambda i: (i + off, 0)
    fix = lambda i: (0, 0)

    in_specs = [
        pl.BlockSpec((block_e, h), blk),
        pl.BlockSpec((block_e, h), ebk),
        pl.BlockSpec((h, h), fix),
        pl.BlockSpec((h, h), fix),
        pl.BlockSpec((1, h), fix),
        pl.BlockSpec((h, 1), fix),
        pl.BlockSpec((1, 1), fix),
    ]
    args = [g, ea, w1c, w2, b2.reshape(1, h), ws, bs.reshape(1, 1)]
    aliases = {}
    if dnew_prev is not None:
        def body(g_ref, ea_ref, w1c_ref, w2_ref, b2_ref, ws_ref, bs_ref,
                 dn_prev, dnew_ref, gated_ref):
            _edge_mlp_body(g_ref, ea_ref, w1c_ref, w2_ref, b2_ref, ws_ref,
                           bs_ref, dnew_ref, gated_ref)

        in_specs.append(pl.BlockSpec(memory_space=pl.ANY))
        args.append(dnew_prev)
        aliases = {7: 0}
    else:
        body = _edge_mlp_body

    return pl.pallas_call(
        body,
        grid=(nb,),
        in_specs=in_specs,
        out_specs=[
            pl.BlockSpec((block_e, h), ebk),
            pl.BlockSpec((block_e, h), blk),
        ],
        out_shape=[
            jax.ShapeDtypeStruct((e, h), jnp.float32),
            jax.ShapeDtypeStruct((eh, h), jnp.float32),
        ],
        input_output_aliases=aliases,
    )(*args)


def _make_scatter(n, e, h, chunk):
    epw = e // _NW
    n_chunks = epw // chunk
    zch = 80
    n_zchunks = n // zch
    mesh = plsc.VectorSubcoreMesh(core_axis_name="c", subcore_axis_name="s")

    @functools.partial(
        pl.kernel,
        out_type=jax.ShapeDtypeStruct((_NC, n, h), jnp.float32),
        mesh=mesh,
        scratch_types=[
            pltpu.VMEM((n_chunks, chunk), jnp.int32),
            pltpu.VMEM((chunk, h), jnp.float32),
            pltpu.VMEM((chunk, h), jnp.float32),
            pltpu.VMEM((zch, h), jnp.float32),
            pltpu.VMEM_SHARED((n, h), jnp.float32),
            pltpu.SemaphoreType.DMA,
            pltpu.SemaphoreType.DMA,
        ],
    )
    def scatter_k(gated_hbm, dst_hbm, out_hbm, idx_v, rv0, rv1, zbuf, acc_sh,
                  sl0, sl1):
        c = lax.axis_index("c")
        s = lax.axis_index("s")
        wid = s * _NC + c
        base = wid * epw
        zero = jnp.zeros((_L,), jnp.float32)

        def zr(r, carry):
            for j in range(h // _L):
                zbuf[r, pl.ds(j * _L, _L)] = zero
            return carry

        lax.fori_loop(0, zch, zr, 0)

        n_mine = (n_zchunks - s + _NS - 1) // _NS

        def zdma(i, carry):
            q = s + i * _NS
            pltpu.sync_copy(zbuf, acc_sh.at[pl.ds(q * zch, zch)])
            return carry

        lax.fori_loop(0, n_mine, zdma, 0)
        pltpu.sync_copy(dst_hbm.at[wid], idx_v)
        plsc.subcore_barrier()

        RV = (rv0, rv1)
        SL = (sl0, sl1)

        def start_load(g, b):
            pltpu.async_copy(
                gated_hbm.at[pl.ds(base + g * chunk, chunk)], RV[b], SL[b])

        def wait_load(b):
            pltpu.make_async_copy(
                gated_hbm.at[pl.ds(base, chunk)], RV[b], SL[b]).wait()

        for b in range(2):
            start_load(b, b)

        def pair(p, carry):
            for b in range(2):
                g = p * 2 + b
                wait_load(b)
                pltpu.sync_copy(RV[b], acc_sh.at[idx_v.at[g]], add=True)

                @pl.when(g + 2 < n_chunks)
                def _():
                    start_load(g + 2, b)

            return carry

        lax.fori_loop(0, n_chunks // 2, pair, 0)
        if n_chunks % 2:
            g = n_chunks - 1
            wait_load(0)
            pltpu.sync_copy(RV[0], acc_sh.at[idx_v.at[g]], add=True)
        plsc.subcore_barrier()

        def odma(i, carry):
            r0 = (s + i * _NS) * zch
            pltpu.sync_copy(acc_sh.at[pl.ds(r0, zch)],
                            out_hbm.at[c, pl.ds(r0, zch)])
            return carry

        lax.fori_loop(0, n_mine, odma, 0)

    return scatter_k


def _update_body(ma_ref, mb_ref, x_ref, u1_ref, c1_ref, gam_ref, bet_ref,
                 u2_ref, c2_ref, out_ref):
    x = x_ref[...]
    u_in = ma_ref[0] + ma_ref[1] + mb_ref[0] + mb_ref[1] + x
    u1 = jnp.maximum(
        jnp.dot(u_in, u1_ref[...], preferred_element_type=jnp.float32)
        + c1_ref[...],
        0.0,
    )
    mean = jnp.mean(u1, axis=0, keepdims=True)
    var = jnp.mean((u1 - mean) ** 2, axis=0, keepdims=True)
    u1n = (u1 - mean) * lax.rsqrt(var + 1e-5) * gam_ref[...] + bet_ref[...]
    out_ref[...] = (
        jnp.dot(u1n, u2_ref[...], preferred_element_type=jnp.float32)
        + c2_ref[...]
        + x
    )


def _update(ma, mb, x, u1, c1, gamma, beta, u2, c2):
    n, h = x.shape
    return pl.pallas_call(
        _update_body,
        out_shape=jax.ShapeDtypeStruct((n, h), jnp.float32),
    )(ma, mb, x, u1, c1.reshape(1, h), gamma.reshape(1, h),
      beta.reshape(1, h), u2, c2.reshape(1, h))


def kernel(x, edge_index, edge_attr, W1, b1, W2, b2, Ws, bs,
           U1, c1, gamma, beta, U2, c2):
    n, h = x.shape
    e = edge_attr.shape[0]
    eh = e // 2
    chunk = 40
    src = edge_index[0]
    dst = edge_index[1]
    srcA = src[:eh].reshape(_NW, -1, chunk)
    dstA = dst[:eh].reshape(_NW, -1, chunk)
    srcB = src[eh:].reshape(_NW, -1, chunk)
    dstB = dst[eh:].reshape(_NW, -1, chunk)
    w1c = W1[2 * h :, :]

    ps, pd = _proj(x, W1, b1)
    gather = _make_gather(n, eh, h, chunk)
    scatter = _make_scatter(n, eh, h, chunk)

    gA = gather(ps, pd, srcA, dstA)
    gB = gather(ps, pd, srcB, dstB)
    dnA, gatedA = _edge_mlp_half(gA, edge_attr, w1c, W2, b2, Ws, bs,
                                 block_e=2000, half=0, dnew_prev=None)
    mA = scatter(gatedA, dstA)
    d_new, gatedB = _edge_mlp_half(gB, edge_attr, w1c, W2, b2, Ws, bs,
                                   block_e=2000, half=1, dnew_prev=dnA)
    mB = scatter(gatedB, dstB)
    feat_out = _update(mA, mB, x, U1, c1, gamma, beta, U2, c2)
    return feat_out, d_new


# speedup vs baseline: 5.2444x; 1.0890x over previous
import functools

import jax
import jax.numpy as jnp
from jax import lax
from jax.experimental import pallas as pl
from jax.experimental.pallas import tpu as pltpu
from jax.experimental.pallas import tpu_sc as plsc

_NC = 2
_NS = 16
_NW = _NC * _NS
_L = 16


def _proj_body(x_ref, w1_ref, b1_ref, ps_ref, pd_ref):
    h = ps_ref.shape[1]
    x = x_ref[...]
    ps_ref[...] = jnp.dot(x, w1_ref[:h, :], preferred_element_type=jnp.float32)
    pd_ref[...] = (
        jnp.dot(x, w1_ref[h : 2 * h, :], preferred_element_type=jnp.float32)
        + b1_ref[...]
    )


def _proj(x, w1, b1):
    n, h = x.shape
    return pl.pallas_call(
        _proj_body,
        out_shape=[
            jax.ShapeDtypeStruct((n, h), jnp.float32),
            jax.ShapeDtypeStruct((n, h), jnp.float32),
        ],
    )(x, w1, b1.reshape(1, h))


def _make_gather(n, e, h, chunk):
    epw = e // _NW
    n_chunks = epw // chunk
    mesh = plsc.VectorSubcoreMesh(core_axis_name="c", subcore_axis_name="s")

    @functools.partial(
        pl.kernel,
        out_type=jax.ShapeDtypeStruct((e, h), jnp.float32),
        mesh=mesh,
        scratch_types=(
            [pltpu.VMEM((n_chunks, chunk), jnp.int32)] * 2
            + [pltpu.VMEM((chunk, h), jnp.float32)] * 6
            + [pltpu.SemaphoreType.DMA] * 4
        ),
    )
    def gather_k(ps_hbm, pd_hbm, src_hbm, dst_hbm, out_hbm,
                 idxs_v, idxd_v, rs0, rs1, rd0, rd1, ro0, ro1,
                 sg0, sg1, so0, so1):
        wid = lax.axis_index("s") * _NC + lax.axis_index("c")
        base = wid * epw
        pltpu.sync_copy(src_hbm.at[wid], idxs_v)
        pltpu.sync_copy(dst_hbm.at[wid], idxd_v)

        RS, RD, RO = (rs0, rs1), (rd0, rd1), (ro0, ro1)
        SG, SO = (sg0, sg1), (so0, so1)

        def start_gather(g, b):
            pltpu.async_copy(ps_hbm.at[idxs_v.at[g]], RS[b], SG[b])
            pltpu.async_copy(pd_hbm.at[idxd_v.at[g]], RD[b], SG[b])

        def wait_gather(b):
            pltpu.make_async_copy(ps_hbm.at[idxs_v.at[0]], RS[b], SG[b]).wait()
            pltpu.make_async_copy(pd_hbm.at[idxd_v.at[0]], RD[b], SG[b]).wait()

        def add_rows(b):
            def radd(r, c2):
                for j in range(h // _L):
                    sl = pl.ds(j * _L, _L)
                    RO[b][r, sl] = RS[b][r, sl] + RD[b][r, sl]
                return c2

            lax.fori_loop(0, chunk, radd, 0)

        def start_out(g, b):
            pltpu.async_copy(RO[b], out_hbm.at[pl.ds(base + g * chunk, chunk)],
                             SO[b])

        def wait_out(b):
            pltpu.make_async_copy(
                RO[b], out_hbm.at[pl.ds(base, chunk)], SO[b]).wait()

        for b in range(2):
            start_gather(b, b)

        def pair(p, carry):
            for b in range(2):
                g = p * 2 + b
                wait_gather(b)

                @pl.when(g >= 2)
                def _():
                    wait_out(b)

                add_rows(b)
                start_out(g, b)

                @pl.when(g + 2 < n_chunks)
                def _():
                    start_gather(g + 2, b)

            return carry

        lax.fori_loop(0, n_chunks // 2, pair, 0)
        if n_chunks % 2:
            g = n_chunks - 1
            wait_gather(0)
            wait_out(0)
            add_rows(0)
            start_out(g, 0)
            wait_out(1)
            wait_out(0)
        else:
            wait_out(0)
            wait_out(1)

    return gather_k


def _edge_mlp_body(g_ref, ea_ref, w1c_ref, w2_ref, b2_ref, ws_ref,
                   bs_ref, dnew_ref, gated_ref):
    g = g_ref[...]
    ea = ea_ref[...]
    h1 = jnp.maximum(
        g + jnp.dot(ea, w1c_ref[...], preferred_element_type=jnp.float32), 0.0
    )
    z = jnp.dot(h1, w2_ref[...], preferred_element_type=jnp.float32) + b2_ref[...]
    msg = z * jax.nn.sigmoid(z)
    dnew_ref[...] = ea + msg
    ew = jax.nn.sigmoid(
        jnp.dot(msg, ws_ref[...], preferred_element_type=jnp.float32) + bs_ref[...]
    )
    gated_ref[...] = msg * ew


def _edge_mlp_half(g, ea, w1c, w2, b2, ws, bs, block_e, e_off, dnew_prev):
    eh, h = g.shape
    e = ea.shape[0]
    nb = eh // block_e
    off = e_off // block_e
    blk = lambda i: (i, 0)
    ebk = lambda i: (i + off, 0)
    fix = lambda i: (0, 0)

    in_specs = [
        pl.BlockSpec((block_e, h), blk),
        pl.BlockSpec((block_e, h), ebk),
        pl.BlockSpec((h, h), fix),
        pl.BlockSpec((h, h), fix),
        pl.BlockSpec((1, h), fix),
        pl.BlockSpec((h, 1), fix),
        pl.BlockSpec((1, 1), fix),
    ]
    args = [g, ea, w1c, w2, b2.reshape(1, h), ws, bs.reshape(1, 1)]
    aliases = {}
    if dnew_prev is not None:
        def body(g_ref, ea_ref, w1c_ref, w2_ref, b2_ref, ws_ref,
                 bs_ref, dn_prev, dnew_ref, gated_ref):
            _edge_mlp_body(g_ref, ea_ref, w1c_ref, w2_ref, b2_ref,
                           ws_ref, bs_ref, dnew_ref, gated_ref)

        in_specs.append(pl.BlockSpec(memory_space=pl.ANY))
        args.append(dnew_prev)
        aliases = {7: 0}
    else:
        body = _edge_mlp_body

    return pl.pallas_call(
        body,
        grid=(nb,),
        in_specs=in_specs,
        out_specs=[
            pl.BlockSpec((block_e, h), ebk),
            pl.BlockSpec((block_e, h), blk),
        ],
        out_shape=[
            jax.ShapeDtypeStruct((e, h), jnp.float32),
            jax.ShapeDtypeStruct((eh, h), jnp.float32),
        ],
        input_output_aliases=aliases,
    )(*args)


def _make_scatter(n, e, h, chunk):
    epw = e // _NW
    n_chunks = epw // chunk
    zch = 80
    n_zchunks = n // zch
    mesh = plsc.VectorSubcoreMesh(core_axis_name="c", subcore_axis_name="s")

    @functools.partial(
        pl.kernel,
        out_type=jax.ShapeDtypeStruct((_NC, n, h), jnp.float32),
        mesh=mesh,
        scratch_types=[
            pltpu.VMEM((n_chunks, chunk), jnp.int32),
            pltpu.VMEM((chunk, h), jnp.float32),
            pltpu.VMEM((chunk, h), jnp.float32),
            pltpu.VMEM((zch, h), jnp.float32),
            pltpu.VMEM_SHARED((n, h), jnp.float32),
            pltpu.SemaphoreType.DMA,
            pltpu.SemaphoreType.DMA,
        ],
    )
    def scatter_k(gated_hbm, dst_hbm, out_hbm, idx_v, rv0, rv1, zbuf, acc_sh,
                  sl0, sl1):
        c = lax.axis_index("c")
        s = lax.axis_index("s")
        wid = s * _NC + c
        base = wid * epw
        zero = jnp.zeros((_L,), jnp.float32)

        def zr(r, carry):
            for j in range(h // _L):
                zbuf[r, pl.ds(j * _L, _L)] = zero
            return carry

        lax.fori_loop(0, zch, zr, 0)

        n_mine = (n_zchunks - s + _NS - 1) // _NS

        def zdma(i, carry):
            q = s + i * _NS
            pltpu.sync_copy(zbuf, acc_sh.at[pl.ds(q * zch, zch)])
            return carry

        lax.fori_loop(0, n_mine, zdma, 0)
        pltpu.sync_copy(dst_hbm.at[wid], idx_v)
        plsc.subcore_barrier()

        RV = (rv0, rv1)
        SL = (sl0, sl1)

        def start_load(g, b):
            pltpu.async_copy(
                gated_hbm.at[pl.ds(base + g * chunk, chunk)], RV[b], SL[b])

        def wait_load(b):
            pltpu.make_async_copy(
                gated_hbm.at[pl.ds(base, chunk)], RV[b], SL[b]).wait()

        for b in range(2):
            start_load(b, b)

        def pair(p, carry):
            for b in range(2):
                g = p * 2 + b
                wait_load(b)
                pltpu.sync_copy(RV[b], acc_sh.at[idx_v.at[g]], add=True)

                @pl.when(g + 2 < n_chunks)
                def _():
                    start_load(g + 2, b)

            return carry

        lax.fori_loop(0, n_chunks // 2, pair, 0)
        if n_chunks % 2:
            g = n_chunks - 1
            wait_load(0)
            pltpu.sync_copy(RV[0], acc_sh.at[idx_v.at[g]], add=True)
        plsc.subcore_barrier()

        def odma(i, carry):
            r0 = (s + i * _NS) * zch
            pltpu.sync_copy(acc_sh.at[pl.ds(r0, zch)],
                            out_hbm.at[c, pl.ds(r0, zch)])
            return carry

        lax.fori_loop(0, n_mine, odma, 0)

    return scatter_k


def _update_body(ma_ref, mb_ref, x_ref, u1_ref, c1_ref, gam_ref, bet_ref,
                 u2_ref, c2_ref, out_ref):
    x = x_ref[...]
    u_in = ma_ref[0] + ma_ref[1] + mb_ref[0] + mb_ref[1] + x
    u1 = jnp.maximum(
        jnp.dot(u_in, u1_ref[...], preferred_element_type=jnp.float32)
        + c1_ref[...],
        0.0,
    )
    mean = jnp.mean(u1, axis=0, keepdims=True)
    var = jnp.mean((u1 - mean) ** 2, axis=0, keepdims=True)
    u1n = (u1 - mean) * lax.rsqrt(var + 1e-5) * gam_ref[...] + bet_ref[...]
    out_ref[...] = (
        jnp.dot(u1n, u2_ref[...], preferred_element_type=jnp.float32)
        + c2_ref[...]
        + x
    )


def _update(ma, mb, x, u1, c1, gamma, beta, u2, c2):
    n, h = x.shape
    return pl.pallas_call(
        _update_body,
        out_shape=jax.ShapeDtypeStruct((n, h), jnp.float32),
    )(ma, mb, x, u1, c1.reshape(1, h), gamma.reshape(1, h),
      beta.reshape(1, h), u2, c2.reshape(1, h))


def kernel(x, edge_index, edge_attr, W1, b1, W2, b2, Ws, bs,
           U1, c1, gamma, beta, U2, c2):
    n, h = x.shape
    e = edge_attr.shape[0]
    chunk = 80
    gran = _NW * chunk
    eA = (e // 2 + gran - 1) // gran * gran
    eB = e - eA
    src = edge_index[0]
    dst = edge_index[1]
    srcA = src[:eA].reshape(_NW, -1, chunk)
    dstA = dst[:eA].reshape(_NW, -1, chunk)
    srcB = src[eA:].reshape(_NW, -1, chunk)
    dstB = dst[eA:].reshape(_NW, -1, chunk)
    w1c = W1[2 * h :, :]

    ps, pd = _proj(x, W1, b1)
    gA = _make_gather(n, eA, h, chunk)(ps, pd, srcA, dstA)
    gB = _make_gather(n, eB, h, chunk)(ps, pd, srcB, dstB)
    block_e = 2560
    dnA, gatedA = _edge_mlp_half(gA, edge_attr, w1c, W2, b2, Ws, bs,
                                 block_e=block_e, e_off=0, dnew_prev=None)
    mA = _make_scatter(n, eA, h, chunk)(gatedA, dstA)
    d_new, gatedB = _edge_mlp_half(gB, edge_attr, w1c, W2, b2, Ws, bs,
                                   block_e=block_e, e_off=eA, dnew_prev=dnA)
    mB = _make_scatter(n, eB, h, chunk)(gatedB, dstB)
    feat_out = _update(mA, mB, x, U1, c1, gamma, beta, U2, c2)
    return feat_out, d_new
